# baseline (speedup 1.0000x reference)
import functools

import jax
import jax.numpy as jnp
from jax import lax
from jax.experimental import pallas as pl
from jax.experimental.pallas import tpu as pltpu
from jax.experimental.pallas import tpu_sc as plsc

V_DIM = 1000000
EMB_DIM = 64
BATCH = 16384
CTX = 20

NC = 2
NS = 16
NW = NC * NS

B_PER_W = BATCH // NW
OUT_PER_CHUNK = 4
ROWS_PER_CHUNK = OUT_PER_CHUNK * CTX
N_CHUNKS = B_PER_W // OUT_PER_CHUNK
NBUF = 4
LANES = 16
PAIR_W = 2 * EMB_DIM
COL_GROUPS = EMB_DIM // LANES
INV_CTX = 1.0 / CTX
OFF_W = 88


def _cbow_body(pair_hbm, off_hbm, table_hbm, out_hbm,
               pair_v, off_v, bufs, out_v, sem0, sem1, sem2, sem3):
    sems = (sem0, sem1, sem2, sem3)
    wid = lax.axis_index("s") * NC + lax.axis_index("c")

    pltpu.sync_copy(pair_hbm.at[wid], pair_v)
    pltpu.sync_copy(off_hbm.at[wid], off_v)

    def start_gather(c, b):
        pltpu.async_copy(table_hbm.at[pair_v.at[c]], bufs.at[b], sems[b])

    def wait_gather(b):
        pltpu.make_async_copy(
            table_hbm.at[pair_v.at[0]], bufs.at[b], sems[b]).wait()

    iota = lax.iota(jnp.int32, LANES)
    col_base = [iota + (g * LANES) for g in range(COL_GROUPS)]

    def reduce_chunk(c, b):
        buf = bufs.at[b]
        off_row = off_v.at[c]
        for j in range(OUT_PER_CHUNK):
            out_base = (c * OUT_PER_CHUNK + j) * EMB_DIM
            acc = [None] * COL_GROUPS
            for k in range(CTX):
                r = j * CTX + k
                rsplat = jnp.full((LANES,), r, jnp.int32)
                half = plsc.load_gather(off_row, [jnp.full((LANES,), r + 1,
                                                          jnp.int32)])
                for g in range(COL_GROUPS):
                    v = plsc.load_gather(buf, [rsplat, half + col_base[g]])
                    acc[g] = v if k == 0 else acc[g] + v
            for g in range(COL_GROUPS):
                out_v[pl.ds(out_base + g * LANES, LANES)] = acc[g] * INV_CTX

    for b in range(NBUF):
        start_gather(b, b)

    @pl.loop(0, N_CHUNKS, step=NBUF)
    def _(cc):
        for b in range(NBUF):
            c = cc + b
            wait_gather(b)
            reduce_chunk(c, b)

            @pl.when(c < N_CHUNKS - NBUF)
            def _():
                start_gather(c + NBUF, b)

    pltpu.sync_copy(out_v, out_hbm.at[pl.ds(wid * B_PER_W * EMB_DIM,
                                            B_PER_W * EMB_DIM)])


@jax.jit
def _cbow_sc(pair_grouped, off_grouped, table_pairs):
    mesh = plsc.VectorSubcoreMesh(core_axis_name="c", subcore_axis_name="s")
    run = pl.kernel(
        _cbow_body,
        out_type=jax.ShapeDtypeStruct((BATCH * EMB_DIM,), jnp.float32),
        mesh=mesh,
        scratch_types=[
            pltpu.VMEM((N_CHUNKS, ROWS_PER_CHUNK), jnp.int32),
            pltpu.VMEM((N_CHUNKS, OFF_W), jnp.int32),
            pltpu.VMEM((NBUF, ROWS_PER_CHUNK, PAIR_W), jnp.float32),
            pltpu.VMEM((B_PER_W * EMB_DIM,), jnp.float32),
            pltpu.SemaphoreType.DMA,
            pltpu.SemaphoreType.DMA,
            pltpu.SemaphoreType.DMA,
            pltpu.SemaphoreType.DMA,
        ],
        compiler_params=pltpu.CompilerParams(needs_layout_passes=False),
    )
    return run(pair_grouped, off_grouped, table_pairs).reshape(BATCH, EMB_DIM)


def kernel(x, emb_table):
    xi = x.astype(jnp.int32)
    pair_grouped = (xi >> 1).reshape(NW, N_CHUNKS, ROWS_PER_CHUNK)
    off_grouped = jnp.pad(
        ((xi & 1) * EMB_DIM).reshape(NW, N_CHUNKS, ROWS_PER_CHUNK),
        ((0, 0), (0, 0), (1, OFF_W - ROWS_PER_CHUNK - 1)))
    table_pairs = emb_table.reshape(V_DIM // 2, PAIR_W)
    return _cbow_sc(pair_grouped, off_grouped, table_pairs)
